# baseline (speedup 1.0000x reference)
import jax
import jax.numpy as jnp
from jax.experimental import pallas as pl
from jax.experimental.pallas import tpu as pltpu

NUM_EXPERTS = 8
TOP_K = 2
HIDDEN = 2048
INTER = 1024
NUM_TOKENS = 2048

TM = 256


def _moe_body(logits_ref, x_ref, w1_ref, w3_ref, w2_ref, out_ref):
    e = pl.program_id(1)

    logits = logits_ref[...]
    m = jnp.max(logits, axis=-1, keepdims=True)
    ex = jnp.exp(logits - m)
    probs = ex / jnp.sum(ex, axis=-1, keepdims=True)
    m1 = jnp.max(probs, axis=-1, keepdims=True)
    eidx = jax.lax.broadcasted_iota(jnp.int32, probs.shape, 1)
    am1 = jnp.argmax(probs, axis=-1, keepdims=True)
    masked = jnp.where(eidx == am1, -jnp.inf, probs)
    am2 = jnp.argmax(masked, axis=-1, keepdims=True)
    gate_e = jnp.where((eidx == am1) | (eidx == am2), probs, 0.0)
    gates = jnp.sum(jnp.where(eidx == e, gate_e, 0.0), axis=-1, keepdims=True)

    x = x_ref[...]
    h1 = jnp.dot(x, w1_ref[0], preferred_element_type=jnp.float32)
    h3 = jnp.dot(x, w3_ref[0], preferred_element_type=jnp.float32)
    h = (h1 * jax.nn.sigmoid(h1)) * h3
    y = jnp.dot(h, w2_ref[0], preferred_element_type=jnp.float32)

    @pl.when(e == 0)
    def _():
        out_ref[...] = gates * y

    @pl.when(e > 0)
    def _():
        out_ref[...] += gates * y


def kernel(hidden_states, router_logits, w1, w3, w2):
    grid = (NUM_TOKENS // TM, NUM_EXPERTS)
    return pl.pallas_call(
        _moe_body,
        grid=grid,
        in_specs=[
            pl.BlockSpec((TM, NUM_EXPERTS), lambda t, e: (t, 0)),
            pl.BlockSpec((TM, HIDDEN), lambda t, e: (t, 0)),
            pl.BlockSpec((1, HIDDEN, INTER), lambda t, e: (e, 0, 0)),
            pl.BlockSpec((1, HIDDEN, INTER), lambda t, e: (e, 0, 0)),
            pl.BlockSpec((1, INTER, HIDDEN), lambda t, e: (e, 0, 0)),
        ],
        out_specs=pl.BlockSpec((TM, HIDDEN), lambda t, e: (t, 0)),
        out_shape=jax.ShapeDtypeStruct((NUM_TOKENS, HIDDEN), jnp.float32),
        compiler_params=pltpu.CompilerParams(
            dimension_semantics=("arbitrary", "arbitrary"),
        ),
    )(router_logits, hidden_states, w1, w3, w2)


# speedup vs baseline: 1.2523x; 1.2523x over previous
import functools

import jax
import jax.numpy as jnp
from jax import lax
from jax.experimental import pallas as pl
from jax.experimental.pallas import tpu as pltpu
from jax.experimental.pallas import tpu_sc as plsc

NUM_EXPERTS = 8
TOP_K = 2
HIDDEN = 2048
INTER = 1024
NUM_TOKENS = 2048

TM = 128
P = NUM_TOKENS * TOP_K + NUM_EXPERTS * TM
NT = P // TM
TR = 128
TL = NUM_TOKENS // TR


def _shift_down(x, s):
    return jnp.concatenate([jnp.zeros((s, x.shape[1]), x.dtype), x[:-s]], axis=0)


def _shift_right(x, s):
    return jnp.concatenate([jnp.zeros((x.shape[0], s), x.dtype), x[:, :-s]], axis=1)


def _routing_body(logits_ref, pos0_ref, pos1_ref, g0_ref, g1_ref, eot_ref):
    planes = [logits_ref[:, :, e] for e in range(NUM_EXPERTS)]
    mx = planes[0]
    for e in range(1, NUM_EXPERTS):
        mx = jnp.maximum(mx, planes[e])
    exps = [jnp.exp(p - mx) for p in planes]
    den = exps[0]
    for e in range(1, NUM_EXPERTS):
        den = den + exps[e]
    probs = [ex / den for ex in exps]

    m1 = probs[0]
    for e in range(1, NUM_EXPERTS):
        m1 = jnp.maximum(m1, probs[e])
    big = jnp.full_like(m1, NUM_EXPERTS)
    am1 = big
    for e in range(NUM_EXPERTS):
        am1 = jnp.minimum(am1, jnp.where(probs[e] == m1, float(e), big))
    maskedp = [jnp.where(am1 == float(e), -jnp.inf, probs[e]) for e in range(NUM_EXPERTS)]
    m2 = maskedp[0]
    for e in range(1, NUM_EXPERTS):
        m2 = jnp.maximum(m2, maskedp[e])
    am2 = big
    for e in range(NUM_EXPERTS):
        am2 = jnp.minimum(am2, jnp.where(maskedp[e] == m2, float(e), big))

    g0_ref[...] = m1
    g1_ref[...] = m2

    base = jnp.int32(0)
    pos0 = jnp.zeros_like(m1)
    pos1 = jnp.zeros_like(m1)
    tile_bases = []
    for e in range(NUM_EXPERTS):
        m = (jnp.where(am1 == float(e), 1.0, 0.0)
             + jnp.where(am2 == float(e), 1.0, 0.0))
        li = m
        s = 1
        while s < TL:
            li = li + _shift_right(li, s)
            s *= 2
        rowsum = li[:, TL - 1:TL]
        ri = rowsum
        s = 1
        while s < TR:
            ri = ri + _shift_down(ri, s)
            s *= 2
        rank_excl = (ri - rowsum) + li - m
        cnt = jnp.sum(m).astype(jnp.int32)
        tile_bases.append(base // TM)
        posv = base.astype(jnp.float32) + rank_excl
        pos0 = pos0 + jnp.where(am1 == float(e), posv, 0.0)
        pos1 = pos1 + jnp.where(am2 == float(e), posv, 0.0)
        base = base + ((cnt + TM - 1) // TM) * TM
    pos0_ref[...] = pos0.astype(jnp.int32)
    pos1_ref[...] = pos1.astype(jnp.int32)

    r8 = lax.broadcasted_iota(jnp.int32, (8, 8), 0)
    c8 = lax.broadcasted_iota(jnp.int32, (8, 8), 1)
    jv = r8 * 8 + c8
    eot = jnp.zeros((8, 8), jnp.int32)
    for e in range(1, NUM_EXPERTS):
        eot = eot + jnp.where(jv >= tile_bases[e], 1, 0)
    eot_ref[...] = eot


def _routing(router_logits):
    logits3 = router_logits.reshape(TR, TL, NUM_EXPERTS)
    outs = pl.pallas_call(
        _routing_body,
        out_shape=(
            jax.ShapeDtypeStruct((TR, TL), jnp.int32),
            jax.ShapeDtypeStruct((TR, TL), jnp.int32),
            jax.ShapeDtypeStruct((TR, TL), jnp.float32),
            jax.ShapeDtypeStruct((TR, TL), jnp.float32),
            jax.ShapeDtypeStruct((8, 8), jnp.int32),
        ),
    )(logits3)
    return outs


def _sc_mesh():
    return plsc.VectorSubcoreMesh(core_axis_name="c", subcore_axis_name="s")

NW = 32
ROWS_W = P // NW
GCH = 16
NCH = ROWS_W // GCH


def _dispatch_body(hid_ref, pos0_ref, pos1_ref, xs_ref,
                   pos0_v, pos1_v, tok_v, buf0, buf1, sem0, sem1):
    cid = lax.axis_index("c")
    sid = lax.axis_index("s")
    wid = sid * 2 + cid
    pltpu.sync_copy(pos0_ref, pos0_v)
    pltpu.sync_copy(pos1_ref, pos1_v)

    def zero_body(i, _):
        tok_v[pl.ds(i * 16, 16)] = jnp.zeros((16,), jnp.int32)
        return 0
    lax.fori_loop(0, P // 16, zero_body, 0)

    def scat_body(c, _):
        vals = c * 16 + lax.iota(jnp.int32, 16)
        plsc.store_scatter(tok_v, [pos0_v[c, :]], vals)
        plsc.store_scatter(tok_v, [pos1_v[c, :]], vals)
        return 0
    lax.fori_loop(0, TR, scat_body, 0)

    base = wid * ROWS_W
    bufs = (buf0, buf1)
    sems = (sem0, sem1)
    copies = [None, None]
    copies[0] = pltpu.async_copy(
        hid_ref.at[tok_v.at[pl.ds(base, GCH)]], bufs[0], sems[0])
    for j in range(NCH):
        b = j % 2
        copies[b].wait()
        if j + 1 < NCH:
            nb = (j + 1) % 2
            copies[nb] = pltpu.async_copy(
                hid_ref.at[tok_v.at[pl.ds(base + (j + 1) * GCH, GCH)]],
                bufs[nb], sems[nb])
        pltpu.sync_copy(bufs[b], xs_ref.at[pl.ds(base + j * GCH, GCH)])


def _dispatch(hidden_states, pos0, pos1):
    k = functools.partial(
        pl.kernel,
        mesh=_sc_mesh(),
        out_type=jax.ShapeDtypeStruct((P, HIDDEN), jnp.float32),
        scratch_types=[
            pltpu.VMEM((TR, TL), jnp.int32),
            pltpu.VMEM((TR, TL), jnp.int32),
            pltpu.VMEM((P,), jnp.int32),
            pltpu.VMEM((GCH, HIDDEN), jnp.float32),
            pltpu.VMEM((GCH, HIDDEN), jnp.float32),
            pltpu.SemaphoreType.DMA,
            pltpu.SemaphoreType.DMA,
        ],
        compiler_params=pltpu.CompilerParams(needs_layout_passes=False),
    )(_dispatch_body)
    return k(hidden_states, pos0, pos1)


def _gemm_body(eot_ref, x_ref, w1_ref, w3_ref, w2_ref, y_ref):
    x = x_ref[...]
    h1 = jnp.dot(x, w1_ref[0], preferred_element_type=jnp.float32)
    h3 = jnp.dot(x, w3_ref[0], preferred_element_type=jnp.float32)
    h = (h1 * jax.nn.sigmoid(h1)) * h3
    y_ref[...] = jnp.dot(h, w2_ref[0], preferred_element_type=jnp.float32)


def _grouped_gemm(x_sorted, eot, w1, w3, w2):
    grid_spec = pltpu.PrefetchScalarGridSpec(
        num_scalar_prefetch=1,
        grid=(NT,),
        in_specs=[
            pl.BlockSpec((TM, HIDDEN), lambda i, eot: (i, 0)),
            pl.BlockSpec((1, HIDDEN, INTER), lambda i, eot: (eot[i], 0, 0)),
            pl.BlockSpec((1, HIDDEN, INTER), lambda i, eot: (eot[i], 0, 0)),
            pl.BlockSpec((1, INTER, HIDDEN), lambda i, eot: (eot[i], 0, 0)),
        ],
        out_specs=pl.BlockSpec((TM, HIDDEN), lambda i, eot: (i, 0)),
    )
    return pl.pallas_call(
        _gemm_body,
        grid_spec=grid_spec,
        out_shape=jax.ShapeDtypeStruct((P, HIDDEN), jnp.float32),
        compiler_params=pltpu.CompilerParams(
            dimension_semantics=("arbitrary",),
        ),
    )(eot, x_sorted, w1, w3, w2)


TOK_W = NUM_TOKENS // NW
DCH = TOK_W // 16


def _combine_body(y_ref, pos0_ref, pos1_ref, g0_ref, g1_ref, out_ref,
                  pos0_v, pos1_v, g0_v, g1_v, buf0, buf1, sem0, sem1):
    cid = lax.axis_index("c")
    sid = lax.axis_index("s")
    wid = sid * 2 + cid
    row0 = wid * DCH
    pltpu.sync_copy(pos0_ref.at[pl.ds(row0, DCH)], pos0_v)
    pltpu.sync_copy(pos1_ref.at[pl.ds(row0, DCH)], pos1_v)
    pltpu.sync_copy(g0_ref.at[pl.ds(row0, DCH)], g0_v)
    pltpu.sync_copy(g1_ref.at[pl.ds(row0, DCH)], g1_v)
    for c in range(DCH):
        cp0 = pltpu.async_copy(y_ref.at[pos0_v[c, :]], buf0, sem0)
        cp1 = pltpu.async_copy(y_ref.at[pos1_v[c, :]], buf1, sem1)
        cp0.wait()
        cp1.wait()
        g0row = g0_v[c, :]
        g1row = g1_v[c, :]
        for r in range(16):
            gs0 = g0row[r]
            gs1 = g1row[r]

            def fma_body(k, _):
                sl = pl.ds(k * 16, 16)
                buf0[r, sl] = gs0 * buf0[r, sl] + gs1 * buf1[r, sl]
                return 0
            lax.fori_loop(0, HIDDEN // 16, fma_body, 0)
        pltpu.sync_copy(
            buf0, out_ref.at[pl.ds(wid * TOK_W + c * 16, 16)])


def _combine(y_sorted, pos0, pos1, g0, g1):
    k = functools.partial(
        pl.kernel,
        mesh=_sc_mesh(),
        out_type=jax.ShapeDtypeStruct((NUM_TOKENS, HIDDEN), jnp.float32),
        scratch_types=[
            pltpu.VMEM((DCH, TL), jnp.int32),
            pltpu.VMEM((DCH, TL), jnp.int32),
            pltpu.VMEM((DCH, TL), jnp.float32),
            pltpu.VMEM((DCH, TL), jnp.float32),
            pltpu.VMEM((16, HIDDEN), jnp.float32),
            pltpu.VMEM((16, HIDDEN), jnp.float32),
            pltpu.SemaphoreType.DMA,
            pltpu.SemaphoreType.DMA,
        ],
        compiler_params=pltpu.CompilerParams(needs_layout_passes=False),
    )(_combine_body)
    return k(y_sorted, pos0, pos1, g0, g1)


def kernel(hidden_states, router_logits, w1, w3, w2):
    pos0, pos1, g0, g1, eot = _routing(router_logits)
    x_sorted = _dispatch(hidden_states, pos0, pos1)
    y_sorted = _grouped_gemm(x_sorted, eot.reshape(64), w1, w3, w2)
    return _combine(y_sorted, pos0, pos1, g0, g1)


# speedup vs baseline: 1.9967x; 1.5944x over previous
import functools

import jax
import jax.numpy as jnp
from jax import lax
from jax.experimental import pallas as pl
from jax.experimental.pallas import tpu as pltpu
from jax.experimental.pallas import tpu_sc as plsc

NUM_EXPERTS = 8
TOP_K = 2
HIDDEN = 2048
INTER = 1024
NUM_TOKENS = 2048

TM = 128
P = NUM_TOKENS * TOP_K + NUM_EXPERTS * TM
NT = P // TM
TR = 128
TL = NUM_TOKENS // TR


def _shift_down(x, s):
    return jnp.concatenate([jnp.zeros((s, x.shape[1]), x.dtype), x[:-s]], axis=0)


def _shift_right(x, s):
    return jnp.concatenate([jnp.zeros((x.shape[0], s), x.dtype), x[:, :-s]], axis=1)


def _routing_body(logits_ref, pos0_ref, pos1_ref, g0_ref, g1_ref, eot_ref,
                  used_ref):
    planes = [logits_ref[:, :, e] for e in range(NUM_EXPERTS)]
    mx = planes[0]
    for e in range(1, NUM_EXPERTS):
        mx = jnp.maximum(mx, planes[e])
    exps = [jnp.exp(p - mx) for p in planes]
    den = exps[0]
    for e in range(1, NUM_EXPERTS):
        den = den + exps[e]
    probs = [ex / den for ex in exps]

    m1 = probs[0]
    for e in range(1, NUM_EXPERTS):
        m1 = jnp.maximum(m1, probs[e])
    big = jnp.full_like(m1, NUM_EXPERTS)
    am1 = big
    for e in range(NUM_EXPERTS):
        am1 = jnp.minimum(am1, jnp.where(probs[e] == m1, float(e), big))
    maskedp = [jnp.where(am1 == float(e), -jnp.inf, probs[e]) for e in range(NUM_EXPERTS)]
    m2 = maskedp[0]
    for e in range(1, NUM_EXPERTS):
        m2 = jnp.maximum(m2, maskedp[e])
    am2 = big
    for e in range(NUM_EXPERTS):
        am2 = jnp.minimum(am2, jnp.where(maskedp[e] == m2, float(e), big))

    g0_ref[...] = m1
    g1_ref[...] = m2

    base = jnp.int32(0)
    pos0 = jnp.zeros_like(m1)
    pos1 = jnp.zeros_like(m1)
    tile_bases = []
    for e in range(NUM_EXPERTS):
        m = (jnp.where(am1 == float(e), 1.0, 0.0)
             + jnp.where(am2 == float(e), 1.0, 0.0))
        li = m
        s = 1
        while s < TL:
            li = li + _shift_right(li, s)
            s *= 2
        rowsum = li[:, TL - 1:TL]
        ri = rowsum
        s = 1
        while s < TR:
            ri = ri + _shift_down(ri, s)
            s *= 2
        rank_excl = (ri - rowsum) + li - m
        cnt = jnp.sum(m).astype(jnp.int32)
        tile_bases.append(base // TM)
        posv = base.astype(jnp.float32) + rank_excl
        pos0 = pos0 + jnp.where(am1 == float(e), posv, 0.0)
        pos1 = pos1 + jnp.where(am2 == float(e), posv, 0.0)
        base = base + ((cnt + TM - 1) // TM) * TM
    pos0_ref[...] = pos0.astype(jnp.int32)
    pos1_ref[...] = pos1.astype(jnp.int32)

    r8 = lax.broadcasted_iota(jnp.int32, (8, 8), 0)
    c8 = lax.broadcasted_iota(jnp.int32, (8, 8), 1)
    jv = r8 * 8 + c8
    eot = jnp.zeros((8, 8), jnp.int32)
    for e in range(1, NUM_EXPERTS):
        eot = eot + jnp.where(jv >= tile_bases[e], 1, 0)
    eot_ref[...] = eot
    used_ref[...] = jnp.where(jv < base // TM, 1, 0)


def _routing(router_logits):
    logits3 = router_logits.reshape(TR, TL, NUM_EXPERTS)
    return pl.pallas_call(
        _routing_body,
        out_shape=(
            jax.ShapeDtypeStruct((TR, TL), jnp.int32),
            jax.ShapeDtypeStruct((TR, TL), jnp.int32),
            jax.ShapeDtypeStruct((TR, TL), jnp.float32),
            jax.ShapeDtypeStruct((TR, TL), jnp.float32),
            jax.ShapeDtypeStruct((8, 8), jnp.int32),
            jax.ShapeDtypeStruct((8, 8), jnp.int32),
        ),
    )(logits3)


def _sc_mesh():
    return plsc.VectorSubcoreMesh(core_axis_name="c", subcore_axis_name="s")

NW = 32
TOK_W = NUM_TOKENS // NW
BCH = 16
BNC = TOK_W // BCH


def _dispatch_body(hid_ref, pos0_ref, pos1_ref, xs_ref,
                   pos0_v, pos1_v, buf0, buf1,
                   rsem0, rsem1, wsem00, wsem01, wsem10, wsem11):
    cid = lax.axis_index("c")
    sid = lax.axis_index("s")
    wid = sid * 2 + cid
    row0 = wid * (TOK_W // TL)
    pltpu.sync_copy(pos0_ref.at[pl.ds(row0, TOK_W // TL)], pos0_v)
    pltpu.sync_copy(pos1_ref.at[pl.ds(row0, TOK_W // TL)], pos1_v)

    bufs = (buf0, buf1)
    rsems = (rsem0, rsem1)
    wsems = ((wsem00, wsem01), (wsem10, wsem11))
    tok0 = wid * TOK_W
    rd = [None] * BNC
    wr = [None] * BNC
    rd[0] = pltpu.async_copy(hid_ref.at[pl.ds(tok0, BCH)], bufs[0], rsems[0])
    for c in range(BNC):
        b = c % 2
        rd[c].wait()
        wr[c] = (
            pltpu.async_copy(bufs[b], xs_ref.at[pos0_v.at[c]], wsems[b][0]),
            pltpu.async_copy(bufs[b], xs_ref.at[pos1_v.at[c]], wsems[b][1]),
        )
        if c + 1 < BNC:
            if c >= 1:
                wr[c - 1][0].wait()
                wr[c - 1][1].wait()
            rd[c + 1] = pltpu.async_copy(
                hid_ref.at[pl.ds(tok0 + (c + 1) * BCH, BCH)],
                bufs[(c + 1) % 2], rsems[(c + 1) % 2])
    wr[BNC - 2][0].wait()
    wr[BNC - 2][1].wait()
    wr[BNC - 1][0].wait()
    wr[BNC - 1][1].wait()


def _dispatch(hidden_states, pos0, pos1):
    k = functools.partial(
        pl.kernel,
        mesh=_sc_mesh(),
        out_type=jax.ShapeDtypeStruct((P, HIDDEN), jnp.float32),
        scratch_types=[
            pltpu.VMEM((TOK_W // TL, TL), jnp.int32),
            pltpu.VMEM((TOK_W // TL, TL), jnp.int32),
            pltpu.VMEM((BCH, HIDDEN), jnp.float32),
            pltpu.VMEM((BCH, HIDDEN), jnp.float32),
            pltpu.SemaphoreType.DMA,
            pltpu.SemaphoreType.DMA,
            pltpu.SemaphoreType.DMA,
            pltpu.SemaphoreType.DMA,
            pltpu.SemaphoreType.DMA,
            pltpu.SemaphoreType.DMA,
        ],
        compiler_params=pltpu.CompilerParams(needs_layout_passes=False),
    )(_dispatch_body)
    return k(hidden_states, pos0, pos1)


def _gemm_body(eot_ref, used_ref, x_ref, w1_ref, w3_ref, w2_ref, y_ref):
    i = pl.program_id(0)

    @pl.when(used_ref[i] > 0)
    def _():
        x = x_ref[...]
        h1 = jnp.dot(x, w1_ref[0], preferred_element_type=jnp.float32)
        h3 = jnp.dot(x, w3_ref[0], preferred_element_type=jnp.float32)
        h = (h1 * jax.nn.sigmoid(h1)) * h3
        y_ref[...] = jnp.dot(h, w2_ref[0], preferred_element_type=jnp.float32)


def _grouped_gemm(x_sorted, eot, used, w1, w3, w2):
    grid_spec = pltpu.PrefetchScalarGridSpec(
        num_scalar_prefetch=2,
        grid=(NT,),
        in_specs=[
            pl.BlockSpec((TM, HIDDEN), lambda i, eot, used: (i, 0)),
            pl.BlockSpec((1, HIDDEN, INTER), lambda i, eot, used: (eot[i], 0, 0)),
            pl.BlockSpec((1, HIDDEN, INTER), lambda i, eot, used: (eot[i], 0, 0)),
            pl.BlockSpec((1, INTER, HIDDEN), lambda i, eot, used: (eot[i], 0, 0)),
        ],
        out_specs=pl.BlockSpec((TM, HIDDEN), lambda i, eot, used: (i, 0)),
    )
    return pl.pallas_call(
        _gemm_body,
        grid_spec=grid_spec,
        out_shape=jax.ShapeDtypeStruct((P, HIDDEN), jnp.float32),
        compiler_params=pltpu.CompilerParams(
            dimension_semantics=("arbitrary",),
        ),
    )(eot, used, x_sorted, w1, w3, w2)


DCH = 8
DNC = TOK_W // DCH


def _combine_body(y_ref, pos0_ref, pos1_ref, g0_ref, g1_ref, out_ref,
                  pos0_v, pos1_v, g0_v, g1_v,
                  a0, b0, a1, b1,
                  ga0, gb0, ga1, gb1, ssem0, ssem1):
    cid = lax.axis_index("c")
    sid = lax.axis_index("s")
    wid = sid * 2 + cid
    row0 = wid * (TOK_W // TL)
    pltpu.sync_copy(pos0_ref.at[pl.ds(row0, TOK_W // TL)], pos0_v)
    pltpu.sync_copy(pos1_ref.at[pl.ds(row0, TOK_W // TL)], pos1_v)
    pltpu.sync_copy(g0_ref.at[pl.ds(row0, TOK_W // TL)], g0_v)
    pltpu.sync_copy(g1_ref.at[pl.ds(row0, TOK_W // TL)], g1_v)

    abufs = (a0, a1)
    bbufs = (b0, b1)
    gsems = ((ga0, gb0), (ga1, gb1))
    ssems = (ssem0, ssem1)

    def start_gathers(c, slot):
        cc, h = c // 2, (c % 2) * DCH
        i0 = pos0_v.at[cc].at[pl.ds(h, DCH)]
        i1 = pos1_v.at[cc].at[pl.ds(h, DCH)]
        return (
            pltpu.async_copy(y_ref.at[i0], abufs[slot], gsems[slot][0]),
            pltpu.async_copy(y_ref.at[i1], bbufs[slot], gsems[slot][1]),
        )

    g = [None] * DNC
    st = [None] * DNC
    g[0] = start_gathers(0, 0)
    for c in range(DNC):
        slot = c % 2
        g[c][0].wait()
        g[c][1].wait()
        if c >= 2:
            st[c - 2].wait()
        if c + 1 < DNC:
            g[c + 1] = start_gathers(c + 1, (c + 1) % 2)
        cc, h = c // 2, (c % 2) * DCH
        g0row = g0_v[cc, :]
        g1row = g1_v[cc, :]
        a = abufs[slot]
        b = bbufs[slot]
        for r in range(DCH):
            gs0 = g0row[h + r]
            gs1 = g1row[h + r]

            def fma_body(k, a=a, b=b, r=r, gs0=gs0, gs1=gs1):
                sl = pl.ds(k * 16, 16)
                a[r, sl] = gs0 * a[r, sl] + gs1 * b[r, sl]

            plsc.parallel_loop(0, HIDDEN // 16, 1, unroll=8)(fma_body)
        st[c] = pltpu.async_copy(
            a, out_ref.at[pl.ds(wid * TOK_W + c * DCH, DCH)], ssems[slot])
    st[DNC - 2].wait()
    st[DNC - 1].wait()


def _combine(y_sorted, pos0, pos1, g0, g1):
    k = functools.partial(
        pl.kernel,
        mesh=_sc_mesh(),
        out_type=jax.ShapeDtypeStruct((NUM_TOKENS, HIDDEN), jnp.float32),
        scratch_types=[
            pltpu.VMEM((TOK_W // TL, TL), jnp.int32),
            pltpu.VMEM((TOK_W // TL, TL), jnp.int32),
            pltpu.VMEM((TOK_W // TL, TL), jnp.float32),
            pltpu.VMEM((TOK_W // TL, TL), jnp.float32),
            pltpu.VMEM((DCH, HIDDEN), jnp.float32),
            pltpu.VMEM((DCH, HIDDEN), jnp.float32),
            pltpu.VMEM((DCH, HIDDEN), jnp.float32),
            pltpu.VMEM((DCH, HIDDEN), jnp.float32),
            pltpu.SemaphoreType.DMA,
            pltpu.SemaphoreType.DMA,
            pltpu.SemaphoreType.DMA,
            pltpu.SemaphoreType.DMA,
            pltpu.SemaphoreType.DMA,
            pltpu.SemaphoreType.DMA,
        ],
        compiler_params=pltpu.CompilerParams(needs_layout_passes=False),
    )(_combine_body)
    return k(y_sorted, pos0, pos1, g0, g1)


def kernel(hidden_states, router_logits, w1, w3, w2):
    pos0, pos1, g0, g1, eot, used = _routing(router_logits)
    x_sorted = _dispatch(hidden_states, pos0, pos1)
    y_sorted = _grouped_gemm(x_sorted, eot.reshape(64), used.reshape(64),
                             w1, w3, w2)
    return _combine(y_sorted, pos0, pos1, g0, g1)
